# Initial kernel scaffold; baseline (speedup 1.0000x reference)
import jax
import jax.numpy as jnp
from jax.experimental import pallas as pl


def kernel(x_drug, edge_drug, x_prot, inc_drug, prot_inc, pos_edge_index, neg_edge_index, Wn_d, We_d, a1, a2, Wp, Wq_d, Wk_p, Wv_p, Wq_p, Wk_d, Wv_d, Wback_d, Wback_p, W1, b1, W2, b2):
    raise NotImplementedError("write your pallas kernel here")


# speedup vs baseline: 2.5003x; 2.5003x over previous
import jax
import jax.numpy as jnp
from jax import lax
from jax.experimental import pallas as pl
from jax.experimental.pallas import tpu as pltpu
from jax.experimental.pallas import tpu_sc as plsc

NC = 2
NS = 16
NW = NC * NS
LL = 16

HD = 128
NDRUG = 50000
NPROT = 50000
ED = 10000
NNZV = 800000
NEV = 200000

SP_E = 10240
SP_N = 51200
R_E, RP_E, NR_E = 10104, 10112, 1
R_N, RP_N, NR_N = 12512, 12544, 4
NNZP = 802816
NEP = 401408
CH = 128


def _mesh():
    return plsc.VectorSubcoreMesh(core_axis_name="c", subcore_axis_name="s")


def _bm(m, cap=1024):
    for c in (1024, 1000, 512, 500, 256, 250, 200, 128, 125, 104, 100, 64,
              56, 50, 40, 32, 25, 16, 8):
        if c <= cap and m % c == 0:
            return c
    return m


def _mm(a, b):
    m, k = a.shape
    _, n = b.shape
    bm = _bm(m)

    def kern(a_ref, b_ref, o_ref):
        o_ref[...] = jnp.dot(a_ref[...], b_ref[...],
                             preferred_element_type=jnp.float32)

    return pl.pallas_call(
        kern,
        grid=(m // bm,),
        in_specs=[pl.BlockSpec((bm, k), lambda i: (i, 0)),
                  pl.BlockSpec((k, n), lambda i: (0, 0))],
        out_specs=pl.BlockSpec((bm, n), lambda i: (i, 0)),
        out_shape=jax.ShapeDtypeStruct((m, n), jnp.float32),
    )(a, b)


def _ew(fn, *arrays):
    m, n = arrays[0].shape
    bm = _bm(m)

    def kern(*refs):
        refs[-1][...] = fn(*[r[...] for r in refs[:-1]])

    return pl.pallas_call(
        kern,
        grid=(m // bm,),
        in_specs=[pl.BlockSpec((bm, n), lambda i: (i, 0))] * len(arrays),
        out_specs=pl.BlockSpec((bm, n), lambda i: (i, 0)),
        out_shape=jax.ShapeDtypeStruct((m, n), jnp.float32),
    )(*arrays)


def _recip_eps(den, sp):
    d = den.reshape(NW, sp // 128, 128)
    r = _ew(lambda *xs: 1.0 / (sum(xs) + 1e-9), *[d[i] for i in range(NW)])
    return r.reshape(sp)


def _recip_deg(den, sp):
    d = den.reshape(NW, sp // 128, 128)
    r = _ew(lambda *xs: 1.0 / jnp.maximum(sum(xs), 1.0),
            *[d[i] for i in range(NW)])
    return r.reshape(sp)


def _zero_rows(buf, nrows, width):
    for i in range(nrows):
        for jj in range(width // LL):
            buf[i, pl.ds(jj * LL, LL)] = jnp.zeros((LL,), jnp.float32)


def _scalar_pass(p_tab, q_tab, aidx, bidx, seg_is_a, sp):
    pn = p_tab.shape[0]
    qn = q_tab.shape[0]
    nnz = aidx.shape[0]
    perw = nnz // NW
    nch = perw // CH

    def body(p_hbm, q_hbm, a_hbm, b_hbm, w_hbm, den_hbm,
             p_v, q_v, ai_v, bi_v, w_v, den_v):
        cid = lax.axis_index("c")
        sid = lax.axis_index("s")
        wid = sid * NC + cid
        pltpu.sync_copy(p_hbm, p_v)
        pltpu.sync_copy(q_hbm, q_v)

        def zf(i, cc):
            den_v[pl.ds(i * LL, LL)] = jnp.zeros((LL,), jnp.float32)
            return cc

        lax.fori_loop(0, sp // LL, zf, 0)

        def chunk(j, cc):
            base = wid * perw + j * CH
            pltpu.sync_copy(a_hbm.at[pl.ds(base, CH)], ai_v)
            pltpu.sync_copy(b_hbm.at[pl.ds(base, CH)], bi_v)

            def inner(k, cc2):
                ia = ai_v[pl.ds(k * LL, LL)]
                ib = bi_v[pl.ds(k * LL, LL)]
                s = plsc.load_gather(p_v, [ia]) + plsc.load_gather(q_v, [ib])
                s = jnp.where(s >= 0.0, s, 0.2 * s)
                wv = jnp.exp(s)
                w_v[pl.ds(k * LL, LL)] = wv
                plsc.addupdate_scatter(den_v, [ia if seg_is_a else ib], wv)
                return cc2

            lax.fori_loop(0, CH // LL, inner, 0)
            pltpu.sync_copy(w_v, w_hbm.at[pl.ds(base, CH)])
            return cc

        lax.fori_loop(0, nch, chunk, 0)
        pltpu.sync_copy(den_v, den_hbm.at[pl.ds(wid * sp, sp)])

    f = pl.kernel(
        body,
        compiler_params=pltpu.CompilerParams(needs_layout_passes=False),
        out_type=(jax.ShapeDtypeStruct((nnz,), jnp.float32),
                  jax.ShapeDtypeStruct((NW * sp,), jnp.float32)),
        mesh=_mesh(),
        scratch_types=[
            pltpu.VMEM((pn,), jnp.float32),
            pltpu.VMEM((qn,), jnp.float32),
            pltpu.VMEM((CH,), jnp.int32),
            pltpu.VMEM((CH,), jnp.int32),
            pltpu.VMEM((CH,), jnp.float32),
            pltpu.VMEM((sp,), jnp.float32),
        ],
    )
    return f(p_tab, q_tab, aidx, bidx)


def _deg_pass(seg, sp):
    nnz = seg.shape[0]
    perw = nnz // NW
    nch = perw // CH

    def body(s_hbm, den_hbm, si_v, den_v):
        cid = lax.axis_index("c")
        sid = lax.axis_index("s")
        wid = sid * NC + cid

        def zf(i, cc):
            den_v[pl.ds(i * LL, LL)] = jnp.zeros((LL,), jnp.float32)
            return cc

        lax.fori_loop(0, sp // LL, zf, 0)
        onev = jnp.ones((LL,), jnp.float32)

        def chunk(j, cc):
            base = wid * perw + j * CH
            pltpu.sync_copy(s_hbm.at[pl.ds(base, CH)], si_v)

            def inner(k, cc2):
                sv = si_v[pl.ds(k * LL, LL)]
                plsc.addupdate_scatter(den_v, [sv], onev)
                return cc2

            lax.fori_loop(0, CH // LL, inner, 0)
            return cc

        lax.fori_loop(0, nch, chunk, 0)
        pltpu.sync_copy(den_v, den_hbm.at[pl.ds(wid * sp, sp)])

    f = pl.kernel(
        body,
        compiler_params=pltpu.CompilerParams(needs_layout_passes=False),
        out_type=jax.ShapeDtypeStruct((NW * sp,), jnp.float32),
        mesh=_mesh(),
        scratch_types=[
            pltpu.VMEM((CH,), jnp.int32),
            pltpu.VMEM((sp,), jnp.float32),
        ],
    )
    return f(seg)


def _coef_pass(w, rg, rs, gidx, sidx):
    use_w = w is not None
    use_rg = rg is not None
    use_rs = rs is not None
    nnz = gidx.shape[0]
    perw = nnz // NW
    nch = perw // CH
    dummy = jnp.zeros((128,), jnp.float32)
    w_in = w if use_w else jnp.zeros((8,), jnp.float32)
    rg_in = rg if use_rg else dummy
    rs_in = rs if use_rs else dummy

    def body(w_hbm, rg_hbm, rs_hbm, g_hbm, s_hbm, c_hbm,
             rg_v, rs_v, gi_v, si_v, w_v, c_v):
        cid = lax.axis_index("c")
        sid = lax.axis_index("s")
        wid = sid * NC + cid
        if use_rg:
            pltpu.sync_copy(rg_hbm, rg_v)
        if use_rs:
            pltpu.sync_copy(rs_hbm, rs_v)

        def chunk(j, cc):
            base = wid * perw + j * CH
            if use_w:
                pltpu.sync_copy(w_hbm.at[pl.ds(base, CH)], w_v)
            if use_rg:
                pltpu.sync_copy(g_hbm.at[pl.ds(base, CH)], gi_v)
            if use_rs:
                pltpu.sync_copy(s_hbm.at[pl.ds(base, CH)], si_v)

            def inner(k, cc2):
                if use_w:
                    cv = w_v[pl.ds(k * LL, LL)]
                else:
                    cv = jnp.ones((LL,), jnp.float32)
                if use_rs:
                    cv = cv * plsc.load_gather(rs_v, [si_v[pl.ds(k * LL, LL)]])
                if use_rg:
                    cv = cv * plsc.load_gather(rg_v, [gi_v[pl.ds(k * LL, LL)]])
                c_v[pl.ds(k * LL, LL)] = cv
                return cc2

            lax.fori_loop(0, CH // LL, inner, 0)
            pltpu.sync_copy(c_v, c_hbm.at[pl.ds(base, CH)])
            return cc

        lax.fori_loop(0, nch, chunk, 0)

    f = pl.kernel(
        body,
        compiler_params=pltpu.CompilerParams(needs_layout_passes=False),
        out_type=jax.ShapeDtypeStruct((nnz,), jnp.float32),
        mesh=_mesh(),
        scratch_types=[
            pltpu.VMEM((rg_in.shape[0],), jnp.float32),
            pltpu.VMEM((rs_in.shape[0],), jnp.float32),
            pltpu.VMEM((CH,), jnp.int32),
            pltpu.VMEM((CH,), jnp.int32),
            pltpu.VMEM((CH,), jnp.float32),
            pltpu.VMEM((CH,), jnp.float32),
        ],
    )
    return f(w_in, rg_in, rs_in, gidx, sidx)


def _row_pass(tab, gidx, sidx, w, nr, r, rp):
    use_w = w is not None
    nnz = gidx.shape[0]
    perw = nnz // NW
    nch = perw // CH
    spans = rp // NS
    w_in = w if use_w else jnp.zeros((8,), jnp.float32)

    def body(t_hbm, g_hbm, s_hbm, w_hbm, out_hbm,
             gi_v, si_v, sc_v, wl_v, rows_v, zb_v, acc_sh, sem):
        cid = lax.axis_index("c")
        sid = lax.axis_index("s")
        wid = sid * NC + cid
        _zero_rows(zb_v, 8, HD)

        for rr in range(nr):
            lo = rr * r

            def zacc(t, cc):
                pltpu.sync_copy(zb_v,
                                acc_sh.at[pl.ds(sid * spans + t * 8, 8)])
                return cc

            lax.fori_loop(0, spans // 8, zacc, 0)
            plsc.subcore_barrier()

            def chunk(j, cc):
                base = wid * perw + j * CH
                pltpu.sync_copy(g_hbm.at[pl.ds(base, CH)], gi_v)
                pltpu.sync_copy(s_hbm.at[pl.ds(base, CH)], si_v)
                if use_w:
                    pltpu.sync_copy(w_hbm.at[pl.ds(base, CH)], wl_v)
                pltpu.async_copy(t_hbm.at[gi_v], rows_v, sem).wait()

                def coef(k, cc2):
                    sv = si_v[pl.ds(k * LL, LL)]
                    inr = (sv >= lo) & (sv < lo + r)
                    sc_v[pl.ds(k * LL, LL)] = jnp.where(inr, sv - lo, r)
                    return cc2

                lax.fori_loop(0, CH // LL, coef, 0)
                if use_w:
                    def scale(k, cc3):
                        cvec = wl_v[pl.ds(k * LL, LL)]
                        for e in range(LL):
                            cs = cvec[e]
                            row = k * LL + e
                            for jj in range(8):
                                rows_v[row, pl.ds(jj * LL, LL)] = (
                                    rows_v[row, pl.ds(jj * LL, LL)] * cs)
                        return cc3

                    lax.fori_loop(0, CH // LL, scale, 0)
                pltpu.sync_copy(rows_v, acc_sh.at[sc_v], add=True)
                return cc

            lax.fori_loop(0, nch, chunk, 0)
            plsc.subcore_barrier()

            def wout(t, cc):
                off = sid * spans + t * 8
                pltpu.sync_copy(
                    acc_sh.at[pl.ds(off, 8)],
                    out_hbm.at[pl.ds((cid * nr + rr) * rp + off, 8)])
                return cc

            lax.fori_loop(0, spans // 8, wout, 0)
            if rr + 1 < nr:
                plsc.subcore_barrier()

    f = pl.kernel(
        body,
        compiler_params=pltpu.CompilerParams(needs_layout_passes=False),
        out_type=jax.ShapeDtypeStruct((NC * nr * rp, HD), jnp.float32),
        mesh=_mesh(),
        scratch_types=[
            pltpu.VMEM((CH,), jnp.int32),
            pltpu.VMEM((CH,), jnp.int32),
            pltpu.VMEM((CH,), jnp.int32),
            pltpu.VMEM((CH,), jnp.float32),
            pltpu.VMEM((CH, HD), jnp.float32),
            pltpu.VMEM((8, HD), jnp.float32),
            pltpu.VMEM_SHARED((rp, HD), jnp.float32),
            pltpu.SemaphoreType.DMA,
        ],
    )
    return f(tab, gidx, sidx, w_in)


def _dot_pass(atab, btab, agidx, bgidx, asegidx, sp, scale):
    ne = agidx.shape[0]
    perw = ne // NW
    nch = perw // CH

    def body(a_hbm, b_hbm, ag_hbm, bg_hbm, as_hbm, w_hbm, den_hbm,
             ag_v, bg_v, as_v, ar_v, br_v, w_v, den_v, sem):
        cid = lax.axis_index("c")
        sid = lax.axis_index("s")
        wid = sid * NC + cid

        def zf(i, cc):
            den_v[pl.ds(i * LL, LL)] = jnp.zeros((LL,), jnp.float32)
            return cc

        lax.fori_loop(0, sp // LL, zf, 0)

        def chunk(j, cc):
            base = wid * perw + j * CH
            pltpu.sync_copy(ag_hbm.at[pl.ds(base, CH)], ag_v)
            pltpu.sync_copy(bg_hbm.at[pl.ds(base, CH)], bg_v)
            pltpu.sync_copy(as_hbm.at[pl.ds(base, CH)], as_v)
            pltpu.async_copy(a_hbm.at[ag_v], ar_v, sem).wait()
            pltpu.async_copy(b_hbm.at[bg_v], br_v, sem).wait()

            def grp(k, cc2):
                rows = k * LL + lax.iota(jnp.int32, LL)

                def col(jj, acc):
                    cj = jnp.full((LL,), jj, jnp.int32)
                    return acc + (plsc.load_gather(ar_v, [rows, cj]) *
                                  plsc.load_gather(br_v, [rows, cj]))

                acc = lax.fori_loop(0, HD, col, jnp.zeros((LL,), jnp.float32))
                wv = jnp.exp(acc * scale)
                w_v[pl.ds(k * LL, LL)] = wv
                plsc.addupdate_scatter(den_v, [as_v[pl.ds(k * LL, LL)]], wv)
                return cc2

            lax.fori_loop(0, CH // LL, grp, 0)
            pltpu.sync_copy(w_v, w_hbm.at[pl.ds(base, CH)])
            return cc

        lax.fori_loop(0, nch, chunk, 0)
        pltpu.sync_copy(den_v, den_hbm.at[pl.ds(wid * sp, sp)])

    f = pl.kernel(
        body,
        compiler_params=pltpu.CompilerParams(needs_layout_passes=False),
        out_type=(jax.ShapeDtypeStruct((ne,), jnp.float32),
                  jax.ShapeDtypeStruct((NW * sp,), jnp.float32)),
        mesh=_mesh(),
        scratch_types=[
            pltpu.VMEM((CH,), jnp.int32),
            pltpu.VMEM((CH,), jnp.int32),
            pltpu.VMEM((CH,), jnp.int32),
            pltpu.VMEM((CH, HD), jnp.float32),
            pltpu.VMEM((CH, HD), jnp.float32),
            pltpu.VMEM((CH,), jnp.float32),
            pltpu.VMEM((sp,), jnp.float32),
            pltpu.SemaphoreType.DMA,
        ],
    )
    return f(atab, btab, agidx, bgidx, asegidx)


def _mlp_pass(gdtab, gptab, agidx, bgidx, b1, w2, b2):
    ne = agidx.shape[0]
    perw = ne // NW
    nch = perw // CH

    def body(gd_hbm, gp_hbm, ag_hbm, bg_hbm, b1_hbm, w2_hbm, b2_hbm, o_hbm,
             ag_v, bg_v, gr_v, pr_v, b1_v, w2_v, b2_v, o_v, sem):
        cid = lax.axis_index("c")
        sid = lax.axis_index("s")
        wid = sid * NC + cid
        pltpu.sync_copy(b1_hbm, b1_v)
        pltpu.sync_copy(w2_hbm, w2_v)
        pltpu.sync_copy(b2_hbm, b2_v)
        b1r = [b1_v[pl.ds(g * LL, LL)] for g in range(HD // LL)]
        w2r = [w2_v[pl.ds(g * LL, LL)] for g in range(HD // LL)]
        b2s = b2_v[pl.ds(0, LL)][0]

        def chunk(j, cc):
            base = wid * perw + j * CH
            pltpu.sync_copy(ag_hbm.at[pl.ds(base, CH)], ag_v)
            pltpu.sync_copy(bg_hbm.at[pl.ds(base, CH)], bg_v)
            pltpu.async_copy(gd_hbm.at[ag_v], gr_v, sem).wait()
            pltpu.async_copy(gp_hbm.at[bg_v], pr_v, sem).wait()

            def grp(k, cc2):
                rows = k * LL + lax.iota(jnp.int32, LL)
                acc = jnp.zeros((LL,), jnp.float32)
                for jj in range(HD):
                    cj = jnp.full((LL,), jj, jnp.int32)
                    v = (plsc.load_gather(gr_v, [rows, cj]) +
                         plsc.load_gather(pr_v, [rows, cj]) +
                         b1r[jj // LL][jj % LL])
                    acc = acc + jnp.maximum(v, 0.0) * w2r[jj // LL][jj % LL]
                o_v[pl.ds(k * LL, LL)] = acc + b2s
                return cc2

            lax.fori_loop(0, CH // LL, grp, 0)
            pltpu.sync_copy(o_v, o_hbm.at[pl.ds(base, CH)])
            return cc

        lax.fori_loop(0, nch, chunk, 0)

    f = pl.kernel(
        body,
        compiler_params=pltpu.CompilerParams(needs_layout_passes=False),
        out_type=jax.ShapeDtypeStruct((ne,), jnp.float32),
        mesh=_mesh(),
        scratch_types=[
            pltpu.VMEM((CH,), jnp.int32),
            pltpu.VMEM((CH,), jnp.int32),
            pltpu.VMEM((CH, HD), jnp.float32),
            pltpu.VMEM((CH, HD), jnp.float32),
            pltpu.VMEM((HD,), jnp.float32),
            pltpu.VMEM((HD,), jnp.float32),
            pltpu.VMEM((LL,), jnp.float32),
            pltpu.VMEM((CH,), jnp.float32),
            pltpu.SemaphoreType.DMA,
        ],
    )
    return f(gdtab, gptab, agidx, bgidx, b1, w2, b2)


def _padcol(v):
    return jnp.pad(v[:, None], ((0, 0), (0, 127)))


def _sum_ranges(part, nr, r, rp, n):
    p = part.reshape(2, nr, rp, HD)[:, :, :r, :].reshape(2, nr * r, HD)
    return p[0, :n], p[1, :n]


def kernel(x_drug, edge_drug, x_prot, inc_drug, prot_inc, pos_edge_index,
           neg_edge_index, Wn_d, We_d, a1, a2, Wp, Wq_d, Wk_p, Wv_p, Wq_p,
           Wk_d, Wv_d, Wback_d, Wback_p, W1, b1, W2, b2):
    scale = float(1.0 / (HD ** 0.5))

    h = _mm(x_drug, Wn_d)
    e1 = _mm(jnp.pad(edge_drug, ((0, 0), (0, 112))),
             jnp.pad(We_d, ((0, 112), (0, 0))))
    hp = _mm(x_prot, Wp)
    u12 = _mm(h, jnp.pad(jnp.stack([a1[:HD], a2[:HD]], 1),
                         ((0, 0), (0, 126))))
    u1 = jnp.pad(u12[:, 0], (0, 48))
    u2 = jnp.pad(u12[:, 1], (0, 48))
    v1 = jnp.pad(_mm(e1, _padcol(a1[HD:]))[:, 0], (0, 112))
    hpad = jnp.pad(h, ((0, 48), (0, 0)))
    hppad = jnp.pad(hp, ((0, 48), (0, 0)))

    npz = NNZP - NNZV
    srcp = jnp.pad(inc_drug[0], (0, npz), constant_values=NDRUG)
    dstp = jnp.pad(inc_drug[1], (0, npz), constant_values=ED)
    psp = jnp.pad(prot_inc[0], (0, npz), constant_values=NPROT)
    pdp = jnp.pad(prot_inc[1], (0, npz), constant_values=ED)

    w1, den1p = _scalar_pass(u1, v1, srcp, dstp, seg_is_a=False, sp=SP_E)
    r1 = _recip_eps(den1p, SP_E)
    c1 = _coef_pass(w1, None, r1, srcp, dstp)
    mpart = _row_pass(hpad, srcp, dstp, c1, NR_E, R_E, RP_E)
    mp0, mp1 = _sum_ranges(mpart, NR_E, R_E, RP_E, RP_E)
    m = _ew(lambda a, b: a + b, mp0, mp1)
    v2 = _mm(m, _padcol(a2[HD:]))[:, 0]

    w2s, den2p = _scalar_pass(u2, v2, srcp, dstp, seg_is_a=True, sp=SP_N)
    r2 = _recip_eps(den2p, SP_N)
    c2 = _coef_pass(w2s, None, r2, dstp, srcp)
    xdpart = _row_pass(m, dstp, srcp, c2, NR_N, R_N, RP_N)
    xd0, xd1 = _sum_ranges(xdpart, NR_N, R_N, RP_N, NDRUG)
    xd = _ew(lambda a, b: jnp.where(a + b > 0, a + b,
                                    jnp.exp(jnp.minimum(a + b, 0.0)) - 1.0),
             xd0, xd1)

    rde = _recip_deg(_deg_pass(pdp, SP_E), SP_E)
    rdn = _recip_deg(_deg_pass(psp, SP_N), SP_N)
    mepart = _row_pass(hppad, psp, pdp, None, NR_E, R_E, RP_E)
    me0, me1 = _sum_ranges(mepart, NR_E, R_E, RP_E, RP_E)
    me = _ew(lambda a, b: a + b, me0, me1)
    cp = _coef_pass(None, rde, rdn, pdp, psp)
    xppart = _row_pass(me, pdp, psp, cp, NR_N, R_N, RP_N)
    xp0, xp1 = _sum_ranges(xppart, NR_N, R_N, RP_N, NPROT)
    xp = _ew(lambda a, b: jnp.maximum(a + b, 0.0), xp0, xp1)

    di = jnp.concatenate([pos_edge_index[0], neg_edge_index[0]])
    pi = jnp.concatenate([pos_edge_index[1], neg_edge_index[1]])
    npad = NEP - di.shape[0]
    dig = jnp.pad(di, (0, npad))
    pig = jnp.pad(pi, (0, npad))
    diseg = jnp.pad(di, (0, npad), constant_values=NDRUG)
    piseg = jnp.pad(pi, (0, npad), constant_values=NPROT)

    qkvd = _mm(xd, jnp.concatenate([Wq_d, Wk_d, Wv_d], 1))
    qd = qkvd[:, :HD]
    kd = qkvd[:, HD:2 * HD]
    vd = qkvd[:, 2 * HD:]
    qkvp = _mm(xp, jnp.concatenate([Wk_p, Wv_p, Wq_p], 1))
    kp = qkvp[:, :HD]
    vp = qkvp[:, HD:2 * HD]
    qp = qkvp[:, 2 * HD:]

    wa, denap = _dot_pass(qd, kp, dig, pig, diseg, SP_N, scale)
    ra = _recip_eps(denap, SP_N)
    ca = _coef_pass(wa, None, ra, pig, diseg)
    x2part = _row_pass(vp, pig, diseg, ca, NR_N, R_N, RP_N)
    a0, a1p = _sum_ranges(x2part, NR_N, R_N, RP_N, NDRUG)
    xd2 = _ew(lambda x, a, b: x + a + b, xd, a0, a1p)

    wb, denbp = _dot_pass(qp, kd, pig, dig, piseg, SP_N, scale)
    rb = _recip_eps(denbp, SP_N)
    cb = _coef_pass(wb, None, rb, dig, piseg)
    y2part = _row_pass(vd, dig, piseg, cb, NR_N, R_N, RP_N)
    b0, b1p = _sum_ranges(y2part, NR_N, R_N, RP_N, NPROT)
    xp2 = _ew(lambda x, a, b: x + a + b, xp, b0, b1p)

    wgd = _mm(Wback_d, W1[:HD])
    wgp = _mm(Wback_p, W1[HD:])
    gd = _mm(xd2, wgd)
    gp = _mm(xp2, wgp)

    logits = _mlp_pass(gd, gp, dig, pig, b1, W2[:, 0], jnp.pad(b2, (0, 15)))
    return logits[:2 * NEV]


# speedup vs baseline: 2.5289x; 1.0114x over previous
import jax
import jax.numpy as jnp
from jax import lax
from jax.experimental import pallas as pl
from jax.experimental.pallas import tpu as pltpu
from jax.experimental.pallas import tpu_sc as plsc

NC = 2
NS = 16
NW = NC * NS
LL = 16

HD = 128
NDRUG = 50000
NPROT = 50000
ED = 10000
NNZV = 800000
NEV = 200000

SP_E = 10240
SP_N = 51200
R_E, RP_E, NR_E = 10104, 10112, 1
R_N, RP_N, NR_N = 12512, 12544, 4
NNZP = 802816
NEP = 401408
CH = 128


def _mesh():
    return plsc.VectorSubcoreMesh(core_axis_name="c", subcore_axis_name="s")


def _bm(m, cap=1024):
    for c in (1024, 1000, 512, 500, 256, 250, 200, 128, 125, 104, 100, 64,
              56, 50, 40, 32, 25, 16, 8):
        if c <= cap and m % c == 0:
            return c
    return m


def _mm(a, b):
    m, k = a.shape
    _, n = b.shape
    bm = _bm(m)

    def kern(a_ref, b_ref, o_ref):
        o_ref[...] = jnp.dot(a_ref[...], b_ref[...],
                             preferred_element_type=jnp.float32)

    return pl.pallas_call(
        kern,
        grid=(m // bm,),
        in_specs=[pl.BlockSpec((bm, k), lambda i: (i, 0)),
                  pl.BlockSpec((k, n), lambda i: (0, 0))],
        out_specs=pl.BlockSpec((bm, n), lambda i: (i, 0)),
        out_shape=jax.ShapeDtypeStruct((m, n), jnp.float32),
    )(a, b)


def _ew(fn, *arrays):
    m, n = arrays[0].shape
    bm = _bm(m)

    def kern(*refs):
        refs[-1][...] = fn(*[r[...] for r in refs[:-1]])

    return pl.pallas_call(
        kern,
        grid=(m // bm,),
        in_specs=[pl.BlockSpec((bm, n), lambda i: (i, 0))] * len(arrays),
        out_specs=pl.BlockSpec((bm, n), lambda i: (i, 0)),
        out_shape=jax.ShapeDtypeStruct((m, n), jnp.float32),
    )(*arrays)


def _recip_eps(den, sp):
    d = den.reshape(NW, sp // 128, 128)
    r = _ew(lambda *xs: 1.0 / (sum(xs) + 1e-9), *[d[i] for i in range(NW)])
    return r.reshape(sp)


def _recip_deg(den, sp):
    d = den.reshape(NW, sp // 128, 128)
    r = _ew(lambda *xs: 1.0 / jnp.maximum(sum(xs), 1.0),
            *[d[i] for i in range(NW)])
    return r.reshape(sp)


def _zero_rows(buf, nrows, width):
    for i in range(nrows):
        for jj in range(width // LL):
            buf[i, pl.ds(jj * LL, LL)] = jnp.zeros((LL,), jnp.float32)


def _scalar_pass(p_tab, q_tab, aidx, bidx, seg_is_a, sp):
    pn = p_tab.shape[0]
    qn = q_tab.shape[0]
    nnz = aidx.shape[0]
    perw = nnz // NW
    nch = perw // CH

    def body(p_hbm, q_hbm, a_hbm, b_hbm, w_hbm, den_hbm,
             p_v, q_v, ai_v, bi_v, w_v, den_v):
        cid = lax.axis_index("c")
        sid = lax.axis_index("s")
        wid = sid * NC + cid
        pltpu.sync_copy(p_hbm, p_v)
        pltpu.sync_copy(q_hbm, q_v)

        def zf(i, cc):
            den_v[pl.ds(i * LL, LL)] = jnp.zeros((LL,), jnp.float32)
            return cc

        lax.fori_loop(0, sp // LL, zf, 0)

        def chunk(j, cc):
            base = wid * perw + j * CH
            pltpu.sync_copy(a_hbm.at[pl.ds(base, CH)], ai_v)
            pltpu.sync_copy(b_hbm.at[pl.ds(base, CH)], bi_v)

            def inner(k, cc2):
                ia = ai_v[pl.ds(k * LL, LL)]
                ib = bi_v[pl.ds(k * LL, LL)]
                s = plsc.load_gather(p_v, [ia]) + plsc.load_gather(q_v, [ib])
                s = jnp.where(s >= 0.0, s, 0.2 * s)
                wv = jnp.exp(s)
                w_v[pl.ds(k * LL, LL)] = wv
                plsc.addupdate_scatter(den_v, [ia if seg_is_a else ib], wv)
                return cc2

            lax.fori_loop(0, CH // LL, inner, 0)
            pltpu.sync_copy(w_v, w_hbm.at[pl.ds(base, CH)])
            return cc

        lax.fori_loop(0, nch, chunk, 0)
        pltpu.sync_copy(den_v, den_hbm.at[pl.ds(wid * sp, sp)])

    f = pl.kernel(
        body,
        compiler_params=pltpu.CompilerParams(needs_layout_passes=False),
        out_type=(jax.ShapeDtypeStruct((nnz,), jnp.float32),
                  jax.ShapeDtypeStruct((NW * sp,), jnp.float32)),
        mesh=_mesh(),
        scratch_types=[
            pltpu.VMEM((pn,), jnp.float32),
            pltpu.VMEM((qn,), jnp.float32),
            pltpu.VMEM((CH,), jnp.int32),
            pltpu.VMEM((CH,), jnp.int32),
            pltpu.VMEM((CH,), jnp.float32),
            pltpu.VMEM((sp,), jnp.float32),
        ],
    )
    return f(p_tab, q_tab, aidx, bidx)


def _deg_pass(seg, sp):
    nnz = seg.shape[0]
    perw = nnz // NW
    nch = perw // CH

    def body(s_hbm, den_hbm, si_v, den_v):
        cid = lax.axis_index("c")
        sid = lax.axis_index("s")
        wid = sid * NC + cid

        def zf(i, cc):
            den_v[pl.ds(i * LL, LL)] = jnp.zeros((LL,), jnp.float32)
            return cc

        lax.fori_loop(0, sp // LL, zf, 0)
        onev = jnp.ones((LL,), jnp.float32)

        def chunk(j, cc):
            base = wid * perw + j * CH
            pltpu.sync_copy(s_hbm.at[pl.ds(base, CH)], si_v)

            def inner(k, cc2):
                sv = si_v[pl.ds(k * LL, LL)]
                plsc.addupdate_scatter(den_v, [sv], onev)
                return cc2

            lax.fori_loop(0, CH // LL, inner, 0)
            return cc

        lax.fori_loop(0, nch, chunk, 0)
        pltpu.sync_copy(den_v, den_hbm.at[pl.ds(wid * sp, sp)])

    f = pl.kernel(
        body,
        compiler_params=pltpu.CompilerParams(needs_layout_passes=False),
        out_type=jax.ShapeDtypeStruct((NW * sp,), jnp.float32),
        mesh=_mesh(),
        scratch_types=[
            pltpu.VMEM((CH,), jnp.int32),
            pltpu.VMEM((sp,), jnp.float32),
        ],
    )
    return f(seg)


def _coef_pass(w, rg, rs, gidx, sidx):
    use_w = w is not None
    use_rg = rg is not None
    use_rs = rs is not None
    nnz = gidx.shape[0]
    perw = nnz // NW
    nch = perw // CH
    dummy = jnp.zeros((128,), jnp.float32)
    w_in = w if use_w else jnp.zeros((8,), jnp.float32)
    rg_in = rg if use_rg else dummy
    rs_in = rs if use_rs else dummy

    def body(w_hbm, rg_hbm, rs_hbm, g_hbm, s_hbm, c_hbm,
             rg_v, rs_v, gi_v, si_v, w_v, c_v):
        cid = lax.axis_index("c")
        sid = lax.axis_index("s")
        wid = sid * NC + cid
        if use_rg:
            pltpu.sync_copy(rg_hbm, rg_v)
        if use_rs:
            pltpu.sync_copy(rs_hbm, rs_v)

        def chunk(j, cc):
            base = wid * perw + j * CH
            if use_w:
                pltpu.sync_copy(w_hbm.at[pl.ds(base, CH)], w_v)
            if use_rg:
                pltpu.sync_copy(g_hbm.at[pl.ds(base, CH)], gi_v)
            if use_rs:
                pltpu.sync_copy(s_hbm.at[pl.ds(base, CH)], si_v)

            def inner(k, cc2):
                if use_w:
                    cv = w_v[pl.ds(k * LL, LL)]
                else:
                    cv = jnp.ones((LL,), jnp.float32)
                if use_rs:
                    cv = cv * plsc.load_gather(rs_v, [si_v[pl.ds(k * LL, LL)]])
                if use_rg:
                    cv = cv * plsc.load_gather(rg_v, [gi_v[pl.ds(k * LL, LL)]])
                c_v[pl.ds(k * LL, LL)] = cv
                return cc2

            lax.fori_loop(0, CH // LL, inner, 0)
            pltpu.sync_copy(c_v, c_hbm.at[pl.ds(base, CH)])
            return cc

        lax.fori_loop(0, nch, chunk, 0)

    f = pl.kernel(
        body,
        compiler_params=pltpu.CompilerParams(needs_layout_passes=False),
        out_type=jax.ShapeDtypeStruct((nnz,), jnp.float32),
        mesh=_mesh(),
        scratch_types=[
            pltpu.VMEM((rg_in.shape[0],), jnp.float32),
            pltpu.VMEM((rs_in.shape[0],), jnp.float32),
            pltpu.VMEM((CH,), jnp.int32),
            pltpu.VMEM((CH,), jnp.int32),
            pltpu.VMEM((CH,), jnp.float32),
            pltpu.VMEM((CH,), jnp.float32),
        ],
    )
    return f(w_in, rg_in, rs_in, gidx, sidx)


def _row_pass(tab, gidx, sidx, w, nr, r, rp):
    CHR = 64
    use_w = w is not None
    nnz = gidx.shape[0]
    perw = nnz // NW
    nch = perw // CHR
    npair = nch // 2
    spans = rp // NS
    w_in = w if use_w else jnp.zeros((8,), jnp.float32)

    def body(t_hbm, g_hbm, s_hbm, w_hbm, out_hbm,
             gi0_v, gi1_v, si0_v, si1_v, sc_v, wl0_v, wl1_v,
             rows0_v, rows1_v, zb_v, acc_sh, sem0, sem1):
        cid = lax.axis_index("c")
        sid = lax.axis_index("s")
        wid = sid * NC + cid
        _zero_rows(zb_v, 8, HD)

        def process(lo, si_v, wl_v, rows_v):
            def coef(k, cc2):
                sv = si_v[pl.ds(k * LL, LL)]
                inr = (sv >= lo) & (sv < lo + r)
                sc_v[pl.ds(k * LL, LL)] = jnp.where(inr, sv - lo, r)
                return cc2

            lax.fori_loop(0, CHR // LL, coef, 0)
            if use_w:
                def scale(k, cc3):
                    cvec = wl_v[pl.ds(k * LL, LL)]
                    for e in range(LL):
                        cs = cvec[e]
                        row = k * LL + e
                        for jj in range(8):
                            rows_v[row, pl.ds(jj * LL, LL)] = (
                                rows_v[row, pl.ds(jj * LL, LL)] * cs)
                    return cc3

                lax.fori_loop(0, CHR // LL, scale, 0)
            pltpu.sync_copy(rows_v, acc_sh.at[sc_v], add=True)

        for rr in range(nr):
            lo = rr * r

            def zacc(t, cc):
                pltpu.sync_copy(zb_v,
                                acc_sh.at[pl.ds(sid * spans + t * 8, 8)])
                return cc

            lax.fori_loop(0, spans // 8, zacc, 0)
            plsc.subcore_barrier()

            def pair(g, cc):
                jb0 = wid * perw + (2 * g) * CHR
                jb1 = jb0 + CHR
                pltpu.sync_copy(g_hbm.at[pl.ds(jb0, CHR)], gi0_v)
                pltpu.sync_copy(s_hbm.at[pl.ds(jb0, CHR)], si0_v)
                if use_w:
                    pltpu.sync_copy(w_hbm.at[pl.ds(jb0, CHR)], wl0_v)
                cp0 = pltpu.async_copy(t_hbm.at[gi0_v], rows0_v, sem0)
                pltpu.sync_copy(g_hbm.at[pl.ds(jb1, CHR)], gi1_v)
                pltpu.sync_copy(s_hbm.at[pl.ds(jb1, CHR)], si1_v)
                if use_w:
                    pltpu.sync_copy(w_hbm.at[pl.ds(jb1, CHR)], wl1_v)
                cp1 = pltpu.async_copy(t_hbm.at[gi1_v], rows1_v, sem1)
                cp0.wait()
                process(lo, si0_v, wl0_v, rows0_v)
                cp1.wait()
                process(lo, si1_v, wl1_v, rows1_v)
                return cc

            lax.fori_loop(0, npair, pair, 0)
            plsc.subcore_barrier()

            def wout(t, cc):
                off = sid * spans + t * 8
                pltpu.sync_copy(
                    acc_sh.at[pl.ds(off, 8)],
                    out_hbm.at[pl.ds((cid * nr + rr) * rp + off, 8)])
                return cc

            lax.fori_loop(0, spans // 8, wout, 0)
            if rr + 1 < nr:
                plsc.subcore_barrier()

    f = pl.kernel(
        body,
        compiler_params=pltpu.CompilerParams(needs_layout_passes=False),
        out_type=jax.ShapeDtypeStruct((NC * nr * rp, HD), jnp.float32),
        mesh=_mesh(),
        scratch_types=[
            pltpu.VMEM((CHR,), jnp.int32),
            pltpu.VMEM((CHR,), jnp.int32),
            pltpu.VMEM((CHR,), jnp.int32),
            pltpu.VMEM((CHR,), jnp.int32),
            pltpu.VMEM((CHR,), jnp.int32),
            pltpu.VMEM((CHR,), jnp.float32),
            pltpu.VMEM((CHR,), jnp.float32),
            pltpu.VMEM((CHR, HD), jnp.float32),
            pltpu.VMEM((CHR, HD), jnp.float32),
            pltpu.VMEM((8, HD), jnp.float32),
            pltpu.VMEM_SHARED((rp, HD), jnp.float32),
            pltpu.SemaphoreType.DMA,
            pltpu.SemaphoreType.DMA,
        ],
    )
    return f(tab, gidx, sidx, w_in)


def _dot_pass(atab, btab, agidx, bgidx, asegidx, sp, scale):
    ne = agidx.shape[0]
    perw = ne // NW
    nch = perw // CH

    def body(a_hbm, b_hbm, ag_hbm, bg_hbm, as_hbm, w_hbm, den_hbm,
             ag_v, bg_v, as_v, ar_v, br_v, w_v, den_v, sem, semb):
        cid = lax.axis_index("c")
        sid = lax.axis_index("s")
        wid = sid * NC + cid

        def zf(i, cc):
            den_v[pl.ds(i * LL, LL)] = jnp.zeros((LL,), jnp.float32)
            return cc

        lax.fori_loop(0, sp // LL, zf, 0)

        def chunk(j, cc):
            base = wid * perw + j * CH
            pltpu.sync_copy(ag_hbm.at[pl.ds(base, CH)], ag_v)
            pltpu.sync_copy(bg_hbm.at[pl.ds(base, CH)], bg_v)
            pltpu.sync_copy(as_hbm.at[pl.ds(base, CH)], as_v)
            cpa = pltpu.async_copy(a_hbm.at[ag_v], ar_v, sem)
            cpb = pltpu.async_copy(b_hbm.at[bg_v], br_v, semb)
            cpa.wait()
            cpb.wait()

            def grp(k, cc2):
                rows = k * LL + lax.iota(jnp.int32, LL)

                def col(jj, acc):
                    cj = jnp.full((LL,), jj, jnp.int32)
                    return acc + (plsc.load_gather(ar_v, [rows, cj]) *
                                  plsc.load_gather(br_v, [rows, cj]))

                acc = lax.fori_loop(0, HD, col, jnp.zeros((LL,), jnp.float32))
                wv = jnp.exp(acc * scale)
                w_v[pl.ds(k * LL, LL)] = wv
                plsc.addupdate_scatter(den_v, [as_v[pl.ds(k * LL, LL)]], wv)
                return cc2

            lax.fori_loop(0, CH // LL, grp, 0)
            pltpu.sync_copy(w_v, w_hbm.at[pl.ds(base, CH)])
            return cc

        lax.fori_loop(0, nch, chunk, 0)
        pltpu.sync_copy(den_v, den_hbm.at[pl.ds(wid * sp, sp)])

    f = pl.kernel(
        body,
        compiler_params=pltpu.CompilerParams(needs_layout_passes=False),
        out_type=(jax.ShapeDtypeStruct((ne,), jnp.float32),
                  jax.ShapeDtypeStruct((NW * sp,), jnp.float32)),
        mesh=_mesh(),
        scratch_types=[
            pltpu.VMEM((CH,), jnp.int32),
            pltpu.VMEM((CH,), jnp.int32),
            pltpu.VMEM((CH,), jnp.int32),
            pltpu.VMEM((CH, HD), jnp.float32),
            pltpu.VMEM((CH, HD), jnp.float32),
            pltpu.VMEM((CH,), jnp.float32),
            pltpu.VMEM((sp,), jnp.float32),
            pltpu.SemaphoreType.DMA,
            pltpu.SemaphoreType.DMA,
        ],
    )
    return f(atab, btab, agidx, bgidx, asegidx)


def _mlp_pass(gdtab, gptab, agidx, bgidx, b1, w2, b2):
    ne = agidx.shape[0]
    perw = ne // NW
    nch = perw // CH

    def body(gd_hbm, gp_hbm, ag_hbm, bg_hbm, b1_hbm, w2_hbm, b2_hbm, o_hbm,
             ag_v, bg_v, gr_v, pr_v, b1_v, w2_v, b2_v, o_v, sem, semb):
        cid = lax.axis_index("c")
        sid = lax.axis_index("s")
        wid = sid * NC + cid
        pltpu.sync_copy(b1_hbm, b1_v)
        pltpu.sync_copy(w2_hbm, w2_v)
        pltpu.sync_copy(b2_hbm, b2_v)
        b1r = [b1_v[pl.ds(g * LL, LL)] for g in range(HD // LL)]
        w2r = [w2_v[pl.ds(g * LL, LL)] for g in range(HD // LL)]
        b2s = b2_v[pl.ds(0, LL)][0]

        def chunk(j, cc):
            base = wid * perw + j * CH
            pltpu.sync_copy(ag_hbm.at[pl.ds(base, CH)], ag_v)
            pltpu.sync_copy(bg_hbm.at[pl.ds(base, CH)], bg_v)
            cpa = pltpu.async_copy(gd_hbm.at[ag_v], gr_v, sem)
            cpb = pltpu.async_copy(gp_hbm.at[bg_v], pr_v, semb)
            cpa.wait()
            cpb.wait()

            def grp(k, cc2):
                rows = k * LL + lax.iota(jnp.int32, LL)
                acc = jnp.zeros((LL,), jnp.float32)
                for jj in range(HD):
                    cj = jnp.full((LL,), jj, jnp.int32)
                    v = (plsc.load_gather(gr_v, [rows, cj]) +
                         plsc.load_gather(pr_v, [rows, cj]) +
                         b1r[jj // LL][jj % LL])
                    acc = acc + jnp.maximum(v, 0.0) * w2r[jj // LL][jj % LL]
                o_v[pl.ds(k * LL, LL)] = acc + b2s
                return cc2

            lax.fori_loop(0, CH // LL, grp, 0)
            pltpu.sync_copy(o_v, o_hbm.at[pl.ds(base, CH)])
            return cc

        lax.fori_loop(0, nch, chunk, 0)

    f = pl.kernel(
        body,
        compiler_params=pltpu.CompilerParams(needs_layout_passes=False),
        out_type=jax.ShapeDtypeStruct((ne,), jnp.float32),
        mesh=_mesh(),
        scratch_types=[
            pltpu.VMEM((CH,), jnp.int32),
            pltpu.VMEM((CH,), jnp.int32),
            pltpu.VMEM((CH, HD), jnp.float32),
            pltpu.VMEM((CH, HD), jnp.float32),
            pltpu.VMEM((HD,), jnp.float32),
            pltpu.VMEM((HD,), jnp.float32),
            pltpu.VMEM((LL,), jnp.float32),
            pltpu.VMEM((CH,), jnp.float32),
            pltpu.SemaphoreType.DMA,
            pltpu.SemaphoreType.DMA,
        ],
    )
    return f(gdtab, gptab, agidx, bgidx, b1, w2, b2)


def _padcol(v):
    return jnp.pad(v[:, None], ((0, 0), (0, 127)))


def _sum_ranges(part, nr, r, rp, n):
    p = part.reshape(2, nr, rp, HD)[:, :, :r, :].reshape(2, nr * r, HD)
    return p[0, :n], p[1, :n]


def kernel(x_drug, edge_drug, x_prot, inc_drug, prot_inc, pos_edge_index,
           neg_edge_index, Wn_d, We_d, a1, a2, Wp, Wq_d, Wk_p, Wv_p, Wq_p,
           Wk_d, Wv_d, Wback_d, Wback_p, W1, b1, W2, b2):
    scale = float(1.0 / (HD ** 0.5))

    h = _mm(x_drug, Wn_d)
    e1 = _mm(jnp.pad(edge_drug, ((0, 0), (0, 112))),
             jnp.pad(We_d, ((0, 112), (0, 0))))
    hp = _mm(x_prot, Wp)
    u12 = _mm(h, jnp.pad(jnp.stack([a1[:HD], a2[:HD]], 1),
                         ((0, 0), (0, 126))))
    u1 = jnp.pad(u12[:, 0], (0, 48))
    u2 = jnp.pad(u12[:, 1], (0, 48))
    v1 = jnp.pad(_mm(e1, _padcol(a1[HD:]))[:, 0], (0, 112))
    hpad = jnp.pad(h, ((0, 48), (0, 0)))
    hppad = jnp.pad(hp, ((0, 48), (0, 0)))

    npz = NNZP - NNZV
    srcp = jnp.pad(inc_drug[0], (0, npz), constant_values=NDRUG)
    dstp = jnp.pad(inc_drug[1], (0, npz), constant_values=ED)
    psp = jnp.pad(prot_inc[0], (0, npz), constant_values=NPROT)
    pdp = jnp.pad(prot_inc[1], (0, npz), constant_values=ED)

    w1, den1p = _scalar_pass(u1, v1, srcp, dstp, seg_is_a=False, sp=SP_E)
    r1 = _recip_eps(den1p, SP_E)
    c1 = _coef_pass(w1, None, r1, srcp, dstp)
    mpart = _row_pass(hpad, srcp, dstp, c1, NR_E, R_E, RP_E)
    mp0, mp1 = _sum_ranges(mpart, NR_E, R_E, RP_E, RP_E)
    m = _ew(lambda a, b: a + b, mp0, mp1)
    v2 = _mm(m, _padcol(a2[HD:]))[:, 0]

    w2s, den2p = _scalar_pass(u2, v2, srcp, dstp, seg_is_a=True, sp=SP_N)
    r2 = _recip_eps(den2p, SP_N)
    c2 = _coef_pass(w2s, None, r2, dstp, srcp)
    xdpart = _row_pass(m, dstp, srcp, c2, NR_N, R_N, RP_N)
    xd0, xd1 = _sum_ranges(xdpart, NR_N, R_N, RP_N, NDRUG)
    xd = _ew(lambda a, b: jnp.where(a + b > 0, a + b,
                                    jnp.exp(jnp.minimum(a + b, 0.0)) - 1.0),
             xd0, xd1)

    rde = _recip_deg(_deg_pass(pdp, SP_E), SP_E)
    rdn = _recip_deg(_deg_pass(psp, SP_N), SP_N)
    mepart = _row_pass(hppad, psp, pdp, None, NR_E, R_E, RP_E)
    me0, me1 = _sum_ranges(mepart, NR_E, R_E, RP_E, RP_E)
    me = _ew(lambda a, b: a + b, me0, me1)
    cp = _coef_pass(None, rde, rdn, pdp, psp)
    xppart = _row_pass(me, pdp, psp, cp, NR_N, R_N, RP_N)
    xp0, xp1 = _sum_ranges(xppart, NR_N, R_N, RP_N, NPROT)
    xp = _ew(lambda a, b: jnp.maximum(a + b, 0.0), xp0, xp1)

    di = jnp.concatenate([pos_edge_index[0], neg_edge_index[0]])
    pi = jnp.concatenate([pos_edge_index[1], neg_edge_index[1]])
    npad = NEP - di.shape[0]
    dig = jnp.pad(di, (0, npad))
    pig = jnp.pad(pi, (0, npad))
    diseg = jnp.pad(di, (0, npad), constant_values=NDRUG)
    piseg = jnp.pad(pi, (0, npad), constant_values=NPROT)

    qkvd = _mm(xd, jnp.concatenate([Wq_d, Wk_d, Wv_d], 1))
    qd = qkvd[:, :HD]
    kd = qkvd[:, HD:2 * HD]
    vd = qkvd[:, 2 * HD:]
    qkvp = _mm(xp, jnp.concatenate([Wk_p, Wv_p, Wq_p], 1))
    kp = qkvp[:, :HD]
    vp = qkvp[:, HD:2 * HD]
    qp = qkvp[:, 2 * HD:]

    wa, denap = _dot_pass(qd, kp, dig, pig, diseg, SP_N, scale)
    ra = _recip_eps(denap, SP_N)
    ca = _coef_pass(wa, None, ra, pig, diseg)
    x2part = _row_pass(vp, pig, diseg, ca, NR_N, R_N, RP_N)
    a0, a1p = _sum_ranges(x2part, NR_N, R_N, RP_N, NDRUG)
    xd2 = _ew(lambda x, a, b: x + a + b, xd, a0, a1p)

    wb, denbp = _dot_pass(qp, kd, pig, dig, piseg, SP_N, scale)
    rb = _recip_eps(denbp, SP_N)
    cb = _coef_pass(wb, None, rb, dig, piseg)
    y2part = _row_pass(vd, dig, piseg, cb, NR_N, R_N, RP_N)
    b0, b1p = _sum_ranges(y2part, NR_N, R_N, RP_N, NPROT)
    xp2 = _ew(lambda x, a, b: x + a + b, xp, b0, b1p)

    wgd = _mm(Wback_d, W1[:HD])
    wgp = _mm(Wback_p, W1[HD:])
    gd = _mm(xd2, wgd)
    gp = _mm(xp2, wgp)

    logits = _mlp_pass(gd, gp, dig, pig, b1, W2[:, 0], jnp.pad(b2, (0, 15)))
    return logits[:2 * NEV]
